# baseline (speedup 1.0000x reference)
import functools

import numpy as np
import jax
import jax.numpy as jnp
from jax import lax
from jax.experimental import pallas as pl
from jax.experimental.pallas import tpu as pltpu
from jax.experimental.pallas import tpu_sc as plsc

N = 10000
E = 320000
H = 32
C = 10
NUM_GRAPHS = 64
NTILE = 32
EPS = np.float32(1e-5)


def _tile_ranges(D):
    W = 240 if D == 128 else 384
    half = E // 2
    nwin = -(-half // W)
    big = nwin % 16
    res = []
    for sc in range(2):
        pos = 0
        for t in range(16):
            nw = (nwin // 16 + 1) if t < big else (nwin // 16)
            start, end = pos, min(pos + nw * W, half)
            res.append((sc * half + start, sc * half + end))
            pos = end
    return res


_RANGES = {D: _tile_ranges(D) for D in (128, 32)}
_CHUNK = {128: 80, 32: 128}
_TILE_OF_EDGE = {}
_TILE_START_MASK = {}
for _D, _rng in _RANGES.items():
    toe = np.zeros(E, np.int32)
    tsm = np.zeros(E, bool)
    for _t, (_s, _e) in enumerate(_rng):
        toe[_s:_e] = _t
        tsm[_s] = True
    _TILE_OF_EDGE[_D] = toe
    _TILE_START_MASK[_D] = tsm


def _make_seg_kernel(D):
    K = _CHUNK[D]
    ranges = _RANGES[D]
    starts_np = [s for s, _ in ranges]
    nch_np = [(e - s) // K for s, e in ranges]
    nvec = 8
    out_rows = N + NTILE + 2 * NTILE

    def body(h_hbm, src_hbm, scidx_hbm, rs_hbm, out_hbm,
             srcv, sciv, rsv, rows, stg, sem):
        cid = lax.axis_index("c")
        sid = lax.axis_index("s")
        wid = cid * 16 + sid
        start = jnp.int32(0)
        nch = jnp.int32(0)
        for t in range(NTILE):
            start = jnp.where(wid == t, jnp.int32(starts_np[t]), start)
            nch = jnp.where(wid == t, jnp.int32(nch_np[t]), nch)

        def chunk(c, carry):
            off = pl.multiple_of(start + c * K, 8)
            pltpu.sync_copy(src_hbm.at[pl.ds(off, K)], srcv)
            pltpu.sync_copy(scidx_hbm.at[pl.ds(off, K)], sciv)
            pltpu.sync_copy(rs_hbm.at[pl.ds(off, K)], rsv)
            pltpu.async_copy(h_hbm.at[srcv], rows, sem).wait()

            def group(j, accs):
                base = j * 16
                rv = rsv[pl.ds(base, 16)]
                for u in range(16):
                    fresh = rv[u] == 1
                    new_accs = []
                    for q in range(nvec):
                        r = rows[base + u, pl.ds(q * 16, 16)]
                        a = jnp.where(fresh, r, accs[q] + r)
                        stg[base + u, pl.ds(q * 16, 16)] = a
                        new_accs.append(a)
                    accs = tuple(new_accs)
                return accs

            accs = lax.fori_loop(0, K // 16, group, carry)
            pltpu.sync_copy(stg, out_hbm.at[sciv])
            return accs

        init = tuple(jnp.zeros((16,), jnp.float32) for _ in range(nvec))
        lax.fori_loop(0, nch, chunk, init)

    return pl.kernel(
        body,
        out_type=jax.ShapeDtypeStruct((out_rows, 128), jnp.float32),
        mesh=plsc.VectorSubcoreMesh(core_axis_name="c", subcore_axis_name="s"),
        scratch_types=[
            pltpu.VMEM((K,), jnp.int32),
            pltpu.VMEM((K,), jnp.int32),
            pltpu.VMEM((K,), jnp.int32),
            pltpu.VMEM((K, 128), jnp.float32),
            pltpu.VMEM((K, 128), jnp.float32),
            pltpu.SemaphoreType.DMA,
        ],
    )


_SEG_KERNELS = {}


def _seg_kernel(D):
    if D not in _SEG_KERNELS:
        _SEG_KERNELS[D] = _make_seg_kernel(D)
    return _SEG_KERNELS[D]


def _edge_plan(dst_s, D):
    toe = jnp.asarray(_TILE_OF_EDGE[D])
    tsm = jnp.asarray(_TILE_START_MASK[D])
    prev_dst = jnp.concatenate([dst_s[:1] - 1, dst_s[:-1]])
    next_dst = jnp.concatenate([dst_s[1:], dst_s[-1:] - 1])
    next_tsm = jnp.concatenate([tsm[1:], jnp.ones((1,), bool)])
    run_start = (dst_s != prev_dst) | tsm
    run_end = (dst_s != next_dst) | next_tsm

    starts = jnp.asarray([s for s, _ in _RANGES[D]], jnp.int32)
    ends = jnp.asarray([e for _, e in _RANGES[D]], jnp.int32)
    first_node = dst_s[starts]
    last_node = dst_s[ends - 1]
    is_first = dst_s == first_node[toe]
    is_last = dst_s == last_node[toe]

    scidx = jnp.where(~run_end, N + toe,
             jnp.where(is_first, N + NTILE + 2 * toe,
              jnp.where(is_last, N + NTILE + 2 * toe + 1, dst_s)))

    single = first_node == last_node
    mnode = jnp.zeros((2 * NTILE,), jnp.int32)
    mnode = mnode.at[0::2].set(first_node)
    mnode = mnode.at[1::2].set(jnp.where(single, N, last_node))
    invalid = mnode >= N
    morder = jnp.argsort(invalid.astype(jnp.int32), stable=True)
    mnode_c = mnode[morder]
    valid_c = mnode_c < N
    same_c = jnp.concatenate([jnp.zeros((1,), bool),
                              (mnode_c[1:] == mnode_c[:-1]) & valid_c[1:]])
    gend = valid_c & jnp.concatenate([mnode_c[1:] != mnode_c[:-1],
                                      jnp.ones((1,), bool)])
    wids = jnp.where(gend, mnode_c, N)
    return (scidx.astype(jnp.int32), run_start.astype(jnp.int32),
            morder.astype(jnp.int32), same_c, wids.astype(jnp.int32))


def _segsum_exact(h_nm, src_s, plan, node_mask, D):
    scidx, rs, morder, same_c, wids = plan
    if h_nm.shape[1] < 128:
        h_nm = jnp.pad(h_nm, ((0, 0), (0, 128 - h_nm.shape[1])))
    out_ext = _seg_kernel(D)(h_nm, src_s, scidx, rs)[:, :D]
    prows = out_ext[N + NTILE:][morder]

    def step(carry, inp):
        same, row = inp
        acc = jnp.where(same, carry + row, row)
        return acc, acc

    _, accs = lax.scan(step, jnp.zeros((D,), jnp.float32), (same_c, prows))
    base = jnp.concatenate([out_ext[:N], jnp.zeros((1, D), jnp.float32)], 0)
    base = base.at[wids].set(accs, mode="drop")
    aggr = jnp.where(node_mask, base[:N], 0.0)
    return aggr


def _l0_body(ht_ref, at_ref, w_ref, b_ref, o_ref):
    t = ht_ref[...] + at_ref[...]
    y = jnp.dot(w_ref[...], t, preferred_element_type=jnp.float32)
    o_ref[...] = jnp.maximum(y + b_ref[...], 0.0)


def _bnmm_body(yt_ref, m_ref, v_ref, g_ref, be_ref, w_ref, b_ref, o_ref):
    r = lax.rsqrt(v_ref[...] + EPS)
    hh = (g_ref[...] * (yt_ref[...] - m_ref[...])) * r + be_ref[...]
    y = jnp.dot(w_ref[...], hh, preferred_element_type=jnp.float32)
    o_ref[...] = jnp.maximum(y + b_ref[...], 0.0)


def _bn_body(yt_ref, m_ref, v_ref, g_ref, be_ref, o_ref):
    r = lax.rsqrt(v_ref[...] + EPS)
    o_ref[...] = (g_ref[...] * (yt_ref[...] - m_ref[...])) * r + be_ref[...]


def _head_body(h_ref, batch_ref, l1t_ref, l1b_ref, clt_ref, clb_ref, o_ref):
    gids = lax.broadcasted_iota(jnp.int32, (NUM_GRAPHS, N), 0)
    onehot = (gids == batch_ref[...]).astype(jnp.float32)
    pooled = jnp.dot(onehot, h_ref[...], preferred_element_type=jnp.float32)
    z = jnp.maximum(jnp.dot(pooled, l1t_ref[...],
                            preferred_element_type=jnp.float32)
                    + l1b_ref[...], 0.0)
    logits = jnp.dot(z, clt_ref[...],
                     preferred_element_type=jnp.float32) + clb_ref[...]
    m = jnp.max(logits, axis=-1, keepdims=True)
    s = logits - m
    lse = jnp.log(jnp.sum(jnp.exp(s), axis=-1, keepdims=True))
    o_ref[...] = s - lse


def _tc(body, out_shape):
    return pl.pallas_call(body, out_shape=jax.ShapeDtypeStruct(out_shape,
                                                               jnp.float32))


def _mlp_chain(tt, at, layers):
    l0 = layers[0]
    yt = _tc(_l0_body, (H, N))(tt, at, l0["W"], l0["b"].reshape(H, 1))
    for li in range(1, len(layers) + 1):
        p = layers[li - 1]
        m = jnp.mean(yt, axis=1, keepdims=True)
        v = jnp.var(yt, axis=1, keepdims=True)
        g = p["gamma"].reshape(H, 1)
        be = p["beta"].reshape(H, 1)
        if li < len(layers):
            nxt = layers[li]
            yt = _tc(_bnmm_body, (H, N))(yt, m, v, g, be, nxt["W"],
                                         nxt["b"].reshape(H, 1))
        else:
            yt = _tc(_bn_body, (H, N))(yt, m, v, g, be)
    return yt


def kernel(x, edge_attr, edge_index, batch, conv_params, lin1_W, lin1_b,
           cls_W, cls_b):
    del edge_attr
    src = edge_index[0]
    dst = edge_index[1]
    perm = jnp.argsort(dst, stable=True)
    src_s = src[perm].astype(jnp.int32)
    dst_s = dst[perm].astype(jnp.int32)
    plans = {D: _edge_plan(dst_s, D) for D in (128, 32)}
    node_mask = jnp.zeros((N, 1), bool).at[dst, 0].set(True)

    h_nm = x
    for ci in range(3):
        D = 128 if ci == 0 else 32
        aggr = _segsum_exact(h_nm, src_s, plans[D], node_mask, D)
        yt = _mlp_chain(h_nm.T, aggr.T, conv_params[ci])
        h_nm = yt.T
    return _tc(_head_body, (NUM_GRAPHS, C))(
        h_nm, batch.reshape(1, N).astype(jnp.int32),
        lin1_W.T, lin1_b.reshape(1, H), cls_W.T, cls_b.reshape(1, C))


# speedup vs baseline: 1.0007x; 1.0007x over previous
import functools

import numpy as np
import jax
import jax.numpy as jnp
from jax import lax
from jax.experimental import pallas as pl
from jax.experimental.pallas import tpu as pltpu
from jax.experimental.pallas import tpu_sc as plsc

N = 10000
E = 320000
H = 32
C = 10
NUM_GRAPHS = 64
NTILE = 32
EPS = np.float32(1e-5)


def _tile_ranges(D):
    W = 240 if D == 128 else 384
    half = E // 2
    nwin = -(-half // W)
    big = nwin % 16
    res = []
    for sc in range(2):
        pos = 0
        for t in range(16):
            nw = (nwin // 16 + 1) if t < big else (nwin // 16)
            start, end = pos, min(pos + nw * W, half)
            res.append((sc * half + start, sc * half + end))
            pos = end
    return res


_RANGES = {D: _tile_ranges(D) for D in (128, 32)}
_CHUNK = {128: 80, 32: 128}
_TILE_OF_EDGE = {}
_TILE_START_MASK = {}
for _D, _rng in _RANGES.items():
    toe = np.zeros(E, np.int32)
    tsm = np.zeros(E, bool)
    for _t, (_s, _e) in enumerate(_rng):
        toe[_s:_e] = _t
        tsm[_s] = True
    _TILE_OF_EDGE[_D] = toe
    _TILE_START_MASK[_D] = tsm


def _make_seg_kernel(D):
    K = _CHUNK[D]
    ranges = _RANGES[D]
    starts_np = [s for s, _ in ranges]
    nch_np = [(e - s) // K for s, e in ranges]
    nvec = D // 16
    out_rows = N + NTILE + 2 * NTILE

    def body(h_hbm, src_hbm, scidx_hbm, rs_hbm, out_hbm,
             srcv, sciv, rsv, rows, stg, sem):
        cid = lax.axis_index("c")
        sid = lax.axis_index("s")
        wid = cid * 16 + sid
        start = jnp.int32(0)
        nch = jnp.int32(0)
        for t in range(NTILE):
            start = jnp.where(wid == t, jnp.int32(starts_np[t]), start)
            nch = jnp.where(wid == t, jnp.int32(nch_np[t]), nch)

        def chunk(c, carry):
            off = pl.multiple_of(start + c * K, 8)
            pltpu.sync_copy(src_hbm.at[pl.ds(off, K)], srcv)
            pltpu.sync_copy(scidx_hbm.at[pl.ds(off, K)], sciv)
            pltpu.sync_copy(rs_hbm.at[pl.ds(off, K)], rsv)
            pltpu.async_copy(h_hbm.at[srcv], rows, sem).wait()

            def group(j, accs):
                base = j * 16
                rv = rsv[pl.ds(base, 16)]
                for u in range(16):
                    fresh = rv[u] == 1
                    new_accs = []
                    for q in range(nvec):
                        r = rows[base + u, pl.ds(q * 16, 16)]
                        a = jnp.where(fresh, r, accs[q] + r)
                        stg[base + u, pl.ds(q * 16, 16)] = a
                        new_accs.append(a)
                    accs = tuple(new_accs)
                return accs

            accs = lax.fori_loop(0, K // 16, group, carry)
            pltpu.sync_copy(stg, out_hbm.at[sciv])
            return accs

        init = tuple(jnp.zeros((16,), jnp.float32) for _ in range(nvec))
        lax.fori_loop(0, nch, chunk, init)

    return pl.kernel(
        body,
        out_type=jax.ShapeDtypeStruct((out_rows, 128), jnp.float32),
        mesh=plsc.VectorSubcoreMesh(core_axis_name="c", subcore_axis_name="s"),
        scratch_types=[
            pltpu.VMEM((K,), jnp.int32),
            pltpu.VMEM((K,), jnp.int32),
            pltpu.VMEM((K,), jnp.int32),
            pltpu.VMEM((K, 128), jnp.float32),
            pltpu.VMEM((K, 128), jnp.float32),
            pltpu.SemaphoreType.DMA,
        ],
    )


_SEG_KERNELS = {}


def _seg_kernel(D):
    if D not in _SEG_KERNELS:
        _SEG_KERNELS[D] = _make_seg_kernel(D)
    return _SEG_KERNELS[D]


def _edge_plan(dst_s, D):
    toe = jnp.asarray(_TILE_OF_EDGE[D])
    tsm = jnp.asarray(_TILE_START_MASK[D])
    prev_dst = jnp.concatenate([dst_s[:1] - 1, dst_s[:-1]])
    next_dst = jnp.concatenate([dst_s[1:], dst_s[-1:] - 1])
    next_tsm = jnp.concatenate([tsm[1:], jnp.ones((1,), bool)])
    run_start = (dst_s != prev_dst) | tsm
    run_end = (dst_s != next_dst) | next_tsm

    starts = jnp.asarray([s for s, _ in _RANGES[D]], jnp.int32)
    ends = jnp.asarray([e for _, e in _RANGES[D]], jnp.int32)
    first_node = dst_s[starts]
    last_node = dst_s[ends - 1]
    is_first = dst_s == first_node[toe]
    is_last = dst_s == last_node[toe]

    scidx = jnp.where(~run_end, N + toe,
             jnp.where(is_first, N + NTILE + 2 * toe,
              jnp.where(is_last, N + NTILE + 2 * toe + 1, dst_s)))

    single = first_node == last_node
    mnode = jnp.zeros((2 * NTILE,), jnp.int32)
    mnode = mnode.at[0::2].set(first_node)
    mnode = mnode.at[1::2].set(jnp.where(single, N, last_node))
    invalid = mnode >= N
    morder = jnp.argsort(invalid.astype(jnp.int32), stable=True)
    mnode_c = mnode[morder]
    valid_c = mnode_c < N
    same_c = jnp.concatenate([jnp.zeros((1,), bool),
                              (mnode_c[1:] == mnode_c[:-1]) & valid_c[1:]])
    gend = valid_c & jnp.concatenate([mnode_c[1:] != mnode_c[:-1],
                                      jnp.ones((1,), bool)])
    wids = jnp.where(gend, mnode_c, N)
    return (scidx.astype(jnp.int32), run_start.astype(jnp.int32),
            morder.astype(jnp.int32), same_c, wids.astype(jnp.int32))


def _segsum_exact(h_nm, src_s, plan, node_mask, D):
    scidx, rs, morder, same_c, wids = plan
    if h_nm.shape[1] < 128:
        h_nm = jnp.pad(h_nm, ((0, 0), (0, 128 - h_nm.shape[1])))
    out_ext = _seg_kernel(D)(h_nm, src_s, scidx, rs)[:, :D]
    prows = out_ext[N + NTILE:][morder]

    def step(carry, inp):
        same, row = inp
        acc = jnp.where(same, carry + row, row)
        return acc, acc

    _, accs = lax.scan(step, jnp.zeros((D,), jnp.float32), (same_c, prows))
    base = jnp.concatenate([out_ext[:N], jnp.zeros((1, D), jnp.float32)], 0)
    base = base.at[wids].set(accs, mode="drop")
    aggr = jnp.where(node_mask, base[:N], 0.0)
    return aggr


def _l0_body(ht_ref, at_ref, w_ref, b_ref, o_ref):
    t = ht_ref[...] + at_ref[...]
    y = jnp.dot(w_ref[...], t, preferred_element_type=jnp.float32)
    o_ref[...] = jnp.maximum(y + b_ref[...], 0.0)


def _bnmm_body(yt_ref, m_ref, v_ref, g_ref, be_ref, w_ref, b_ref, o_ref):
    r = lax.rsqrt(v_ref[...] + EPS)
    hh = (g_ref[...] * (yt_ref[...] - m_ref[...])) * r + be_ref[...]
    y = jnp.dot(w_ref[...], hh, preferred_element_type=jnp.float32)
    o_ref[...] = jnp.maximum(y + b_ref[...], 0.0)


def _bn_body(yt_ref, m_ref, v_ref, g_ref, be_ref, o_ref):
    r = lax.rsqrt(v_ref[...] + EPS)
    o_ref[...] = (g_ref[...] * (yt_ref[...] - m_ref[...])) * r + be_ref[...]


def _head_body(h_ref, batch_ref, l1t_ref, l1b_ref, clt_ref, clb_ref, o_ref):
    gids = lax.broadcasted_iota(jnp.int32, (NUM_GRAPHS, N), 0)
    onehot = (gids == batch_ref[...]).astype(jnp.float32)
    pooled = jnp.dot(onehot, h_ref[...], preferred_element_type=jnp.float32)
    z = jnp.maximum(jnp.dot(pooled, l1t_ref[...],
                            preferred_element_type=jnp.float32)
                    + l1b_ref[...], 0.0)
    logits = jnp.dot(z, clt_ref[...],
                     preferred_element_type=jnp.float32) + clb_ref[...]
    m = jnp.max(logits, axis=-1, keepdims=True)
    s = logits - m
    lse = jnp.log(jnp.sum(jnp.exp(s), axis=-1, keepdims=True))
    o_ref[...] = s - lse


def _tc(body, out_shape):
    return pl.pallas_call(body, out_shape=jax.ShapeDtypeStruct(out_shape,
                                                               jnp.float32))


def _mlp_chain(tt, at, layers):
    l0 = layers[0]
    yt = _tc(_l0_body, (H, N))(tt, at, l0["W"], l0["b"].reshape(H, 1))
    for li in range(1, len(layers) + 1):
        p = layers[li - 1]
        m = jnp.mean(yt, axis=1, keepdims=True)
        v = jnp.var(yt, axis=1, keepdims=True)
        g = p["gamma"].reshape(H, 1)
        be = p["beta"].reshape(H, 1)
        if li < len(layers):
            nxt = layers[li]
            yt = _tc(_bnmm_body, (H, N))(yt, m, v, g, be, nxt["W"],
                                         nxt["b"].reshape(H, 1))
        else:
            yt = _tc(_bn_body, (H, N))(yt, m, v, g, be)
    return yt


def kernel(x, edge_attr, edge_index, batch, conv_params, lin1_W, lin1_b,
           cls_W, cls_b):
    del edge_attr
    src = edge_index[0]
    dst = edge_index[1]
    perm = jnp.argsort(dst, stable=True)
    src_s = src[perm].astype(jnp.int32)
    dst_s = dst[perm].astype(jnp.int32)
    plans = {D: _edge_plan(dst_s, D) for D in (128, 32)}
    node_mask = jnp.zeros((N, 1), bool).at[dst, 0].set(True)

    h_nm = x
    for ci in range(3):
        D = 128 if ci == 0 else 32
        aggr = _segsum_exact(h_nm, src_s, plans[D], node_mask, D)
        yt = _mlp_chain(h_nm.T, aggr.T, conv_params[ci])
        h_nm = yt.T
    return _tc(_head_body, (NUM_GRAPHS, C))(
        h_nm, batch.reshape(1, N).astype(jnp.int32),
        lin1_W.T, lin1_b.reshape(1, H), cls_W.T, cls_b.reshape(1, C))


# speedup vs baseline: 1.1258x; 1.1250x over previous
import functools

import numpy as np
import jax
import jax.numpy as jnp
from jax import lax
from jax.experimental import pallas as pl
from jax.experimental.pallas import tpu as pltpu
from jax.experimental.pallas import tpu_sc as plsc

N = 10000
E = 320000
H = 32
C = 10
NUM_GRAPHS = 64
NTILE = 32
EPS = np.float32(1e-5)


def _tile_ranges(D):
    W = 240 if D == 128 else 384
    half = E // 2
    nwin = -(-half // W)
    big = nwin % 16
    res = []
    for sc in range(2):
        pos = 0
        for t in range(16):
            nw = (nwin // 16 + 1) if t < big else (nwin // 16)
            start, end = pos, min(pos + nw * W, half)
            res.append((sc * half + start, sc * half + end))
            pos = end
    return res


_RANGES = {D: _tile_ranges(D) for D in (128, 32)}
_CHUNK = {128: 80, 32: 128}
_TILE_OF_EDGE = {}
_TILE_START_MASK = {}
for _D, _rng in _RANGES.items():
    toe = np.zeros(E, np.int32)
    tsm = np.zeros(E, bool)
    for _t, (_s, _e) in enumerate(_rng):
        toe[_s:_e] = _t
        tsm[_s] = True
    _TILE_OF_EDGE[_D] = toe
    _TILE_START_MASK[_D] = tsm


def _make_seg_kernel(D):
    K = _CHUNK[D]
    ranges = _RANGES[D]
    s0_np = [s // K for s, _ in ranges]
    nch_np = [(e - s) // K for s, e in ranges]
    nvec = D // 16
    out_rows = N + NTILE + 2 * NTILE

    def body(h_hbm, pack_hbm, sci_hbm, out_hbm,
             pb0, pb1, sc0, sc1, rw0, rw1, sg0, sg1, ps0, ps1, gs0, gs1):
        packs = (pb0, pb1)
        scis = (sc0, sc1)
        rows = (rw0, rw1)
        stgs = (sg0, sg1)
        psems = (ps0, ps1)
        gsems = (gs0, gs1)
        cid = lax.axis_index("c")
        sid = lax.axis_index("s")
        wid = cid * 16 + sid
        s0 = jnp.int32(0)
        nch = jnp.int32(0)
        for t in range(NTILE):
            s0 = jnp.where(wid == t, jnp.int32(s0_np[t]), s0)
            nch = jnp.where(wid == t, jnp.int32(nch_np[t]), nch)

        def issue_pack(g, b):
            off = pl.multiple_of(g * 2 * K, 8)
            off2 = pl.multiple_of(g * K, 8)
            pltpu.async_copy(pack_hbm.at[pl.ds(off, 2 * K)], packs[b], psems[b])
            pltpu.async_copy(sci_hbm.at[pl.ds(off2, K)], scis[b], psems[b])

        def wait_pack(b):
            pltpu.make_async_copy(pack_hbm.at[pl.ds(0, 2 * K)], packs[b],
                                  psems[b]).wait()
            pltpu.make_async_copy(sci_hbm.at[pl.ds(0, K)], scis[b],
                                  psems[b]).wait()

        def issue_gather(b):
            pltpu.async_copy(h_hbm.at[packs[b].at[pl.ds(0, K)]], rows[b],
                             gsems[b])

        def wait_gather(b):
            pltpu.make_async_copy(h_hbm.at[pl.ds(0, K)], rows[b],
                                  gsems[b]).wait()

        def compute(b, accs):
            pb = packs[b]
            rw = rows[b]
            sg = stgs[b]

            def group(j, accs):
                base = j * 16
                rv = pb[pl.ds(K + base, 16)]
                for u in range(16):
                    fresh = rv[u] == 1
                    new_accs = []
                    for q in range(nvec):
                        r = rw[base + u, pl.ds(q * 16, 16)]
                        a = jnp.where(fresh, r, accs[q] + r)
                        sg[base + u, pl.ds(q * 16, 16)] = a
                        new_accs.append(a)
                    accs = tuple(new_accs)
                return accs

            return lax.fori_loop(0, K // 16, group, accs)

        def slot(c, b, accs):
            @pl.when(c + 1 < nch)
            def _():
                issue_pack(s0 + c + 1, b ^ 1)
            wait_gather(b)
            accs = compute(b, accs)
            pltpu.sync_copy(stgs[b], out_hbm.at[scis[b]])

            @pl.when(c + 1 < nch)
            def _():
                wait_pack(b ^ 1)
                issue_gather(b ^ 1)
            return accs

        issue_pack(s0, 0)
        wait_pack(0)
        issue_gather(0)

        def pair(c2, accs):
            accs = slot(2 * c2, 0, accs)
            accs = slot(2 * c2 + 1, 1, accs)
            return accs

        accs = lax.fori_loop(0, nch // 2, pair,
                             tuple(jnp.zeros((16,), jnp.float32)
                                   for _ in range(nvec)))
        odd = (nch % 2) == 1

        @pl.when(odd)
        def _():
            e = nch - 1
            wait_gather(0)
            a2 = compute(0, accs)
            del a2
            pltpu.sync_copy(stgs[0], out_hbm.at[scis[0]])

    return pl.kernel(
        body,
        out_type=jax.ShapeDtypeStruct((out_rows, 128), jnp.float32),
        mesh=plsc.VectorSubcoreMesh(core_axis_name="c", subcore_axis_name="s"),
        scratch_types=[
            pltpu.VMEM((2 * K,), jnp.int32),
            pltpu.VMEM((2 * K,), jnp.int32),
            pltpu.VMEM((K,), jnp.int32),
            pltpu.VMEM((K,), jnp.int32),
            pltpu.VMEM((K, 128), jnp.float32),
            pltpu.VMEM((K, 128), jnp.float32),
            pltpu.VMEM((K, 128), jnp.float32),
            pltpu.VMEM((K, 128), jnp.float32),
            pltpu.SemaphoreType.DMA,
            pltpu.SemaphoreType.DMA,
            pltpu.SemaphoreType.DMA,
            pltpu.SemaphoreType.DMA,
        ],
    )


_SEG_KERNELS = {}


def _seg_kernel(D):
    if D not in _SEG_KERNELS:
        _SEG_KERNELS[D] = _make_seg_kernel(D)
    return _SEG_KERNELS[D]


def _edge_plan(dst_s, D):
    toe = jnp.asarray(_TILE_OF_EDGE[D])
    tsm = jnp.asarray(_TILE_START_MASK[D])
    prev_dst = jnp.concatenate([dst_s[:1] - 1, dst_s[:-1]])
    next_dst = jnp.concatenate([dst_s[1:], dst_s[-1:] - 1])
    next_tsm = jnp.concatenate([tsm[1:], jnp.ones((1,), bool)])
    run_start = (dst_s != prev_dst) | tsm
    run_end = (dst_s != next_dst) | next_tsm

    starts = jnp.asarray([s for s, _ in _RANGES[D]], jnp.int32)
    ends = jnp.asarray([e for _, e in _RANGES[D]], jnp.int32)
    first_node = dst_s[starts]
    last_node = dst_s[ends - 1]
    is_first = dst_s == first_node[toe]
    is_last = dst_s == last_node[toe]

    scidx = jnp.where(~run_end, N + toe,
             jnp.where(is_first, N + NTILE + 2 * toe,
              jnp.where(is_last, N + NTILE + 2 * toe + 1, dst_s)))

    single = first_node == last_node
    mnode = jnp.zeros((2 * NTILE,), jnp.int32)
    mnode = mnode.at[0::2].set(first_node)
    mnode = mnode.at[1::2].set(jnp.where(single, N, last_node))
    invalid = mnode >= N
    morder = jnp.argsort(invalid.astype(jnp.int32), stable=True)
    mnode_c = mnode[morder]
    valid_c = mnode_c < N
    same_c = jnp.concatenate([jnp.zeros((1,), bool),
                              (mnode_c[1:] == mnode_c[:-1]) & valid_c[1:]])
    gend = valid_c & jnp.concatenate([mnode_c[1:] != mnode_c[:-1],
                                      jnp.ones((1,), bool)])
    wids = jnp.where(gend, mnode_c, N)
    return (scidx.astype(jnp.int32), run_start.astype(jnp.int32),
            morder.astype(jnp.int32), same_c, wids.astype(jnp.int32))


def _pack_plan(src_s, plan, D):
    scidx, rs, _, _, _ = plan
    K = _CHUNK[D]
    pack = jnp.stack([src_s.reshape(-1, K), rs.reshape(-1, K)],
                     axis=1).reshape(-1)
    return pack, scidx


def _segsum_exact(h_nm, src_s, plan, node_mask, D):
    scidx, rs, morder, same_c, wids = plan
    if h_nm.shape[1] < 128:
        h_nm = jnp.pad(h_nm, ((0, 0), (0, 128 - h_nm.shape[1])))
    pack, sci = _pack_plan(src_s, plan, D)
    out_ext = _seg_kernel(D)(h_nm, pack, sci)[:, :D]
    prows = out_ext[N + NTILE:][morder]

    def step(carry, inp):
        same, row = inp
        acc = jnp.where(same, carry + row, row)
        return acc, acc

    _, accs = lax.scan(step, jnp.zeros((D,), jnp.float32), (same_c, prows))
    base = jnp.concatenate([out_ext[:N], jnp.zeros((1, D), jnp.float32)], 0)
    base = base.at[wids].set(accs, mode="drop")
    aggr = jnp.where(node_mask, base[:N], 0.0)
    return aggr


def _l0_body(ht_ref, at_ref, w_ref, b_ref, o_ref):
    t = ht_ref[...] + at_ref[...]
    y = jnp.dot(w_ref[...], t, preferred_element_type=jnp.float32)
    o_ref[...] = jnp.maximum(y + b_ref[...], 0.0)


def _bnmm_body(yt_ref, m_ref, v_ref, g_ref, be_ref, w_ref, b_ref, o_ref):
    r = lax.rsqrt(v_ref[...] + EPS)
    hh = (g_ref[...] * (yt_ref[...] - m_ref[...])) * r + be_ref[...]
    y = jnp.dot(w_ref[...], hh, preferred_element_type=jnp.float32)
    o_ref[...] = jnp.maximum(y + b_ref[...], 0.0)


def _bn_body(yt_ref, m_ref, v_ref, g_ref, be_ref, o_ref):
    r = lax.rsqrt(v_ref[...] + EPS)
    o_ref[...] = (g_ref[...] * (yt_ref[...] - m_ref[...])) * r + be_ref[...]


def _head_body(h_ref, batch_ref, l1t_ref, l1b_ref, clt_ref, clb_ref, o_ref):
    gids = lax.broadcasted_iota(jnp.int32, (NUM_GRAPHS, N), 0)
    onehot = (gids == batch_ref[...]).astype(jnp.float32)
    pooled = jnp.dot(onehot, h_ref[...], preferred_element_type=jnp.float32)
    z = jnp.maximum(jnp.dot(pooled, l1t_ref[...],
                            preferred_element_type=jnp.float32)
                    + l1b_ref[...], 0.0)
    logits = jnp.dot(z, clt_ref[...],
                     preferred_element_type=jnp.float32) + clb_ref[...]
    m = jnp.max(logits, axis=-1, keepdims=True)
    s = logits - m
    lse = jnp.log(jnp.sum(jnp.exp(s), axis=-1, keepdims=True))
    o_ref[...] = s - lse


def _tc(body, out_shape):
    return pl.pallas_call(body, out_shape=jax.ShapeDtypeStruct(out_shape,
                                                               jnp.float32))


def _mlp_chain(tt, at, layers):
    l0 = layers[0]
    yt = _tc(_l0_body, (H, N))(tt, at, l0["W"], l0["b"].reshape(H, 1))
    for li in range(1, len(layers) + 1):
        p = layers[li - 1]
        m = jnp.mean(yt, axis=1, keepdims=True)
        v = jnp.var(yt, axis=1, keepdims=True)
        g = p["gamma"].reshape(H, 1)
        be = p["beta"].reshape(H, 1)
        if li < len(layers):
            nxt = layers[li]
            yt = _tc(_bnmm_body, (H, N))(yt, m, v, g, be, nxt["W"],
                                         nxt["b"].reshape(H, 1))
        else:
            yt = _tc(_bn_body, (H, N))(yt, m, v, g, be)
    return yt


def kernel(x, edge_attr, edge_index, batch, conv_params, lin1_W, lin1_b,
           cls_W, cls_b):
    del edge_attr
    src = edge_index[0]
    dst = edge_index[1]
    perm = jnp.argsort(dst, stable=True)
    src_s = src[perm].astype(jnp.int32)
    dst_s = dst[perm].astype(jnp.int32)
    plans = {D: _edge_plan(dst_s, D) for D in (128, 32)}
    node_mask = jnp.zeros((N, 1), bool).at[dst, 0].set(True)

    h_nm = x
    for ci in range(3):
        D = 128 if ci == 0 else 32
        aggr = _segsum_exact(h_nm, src_s, plans[D], node_mask, D)
        yt = _mlp_chain(h_nm.T, aggr.T, conv_params[ci])
        h_nm = yt.T
    return _tc(_head_body, (NUM_GRAPHS, C))(
        h_nm, batch.reshape(1, N).astype(jnp.int32),
        lin1_W.T, lin1_b.reshape(1, H), cls_W.T, cls_b.reshape(1, C))


# speedup vs baseline: 1.1266x; 1.0007x over previous
import functools

import numpy as np
import jax
import jax.numpy as jnp
from jax import lax
from jax.experimental import pallas as pl
from jax.experimental.pallas import tpu as pltpu
from jax.experimental.pallas import tpu_sc as plsc

N = 10000
E = 320000
H = 32
C = 10
NUM_GRAPHS = 64
NTILE = 32
EPS = np.float32(1e-5)


def _tile_ranges(D):
    W = 240 if D == 128 else 384
    half = E // 2
    nwin = -(-half // W)
    big = nwin % 16
    res = []
    for sc in range(2):
        pos = 0
        for t in range(16):
            nw = (nwin // 16 + 1) if t < big else (nwin // 16)
            start, end = pos, min(pos + nw * W, half)
            res.append((sc * half + start, sc * half + end))
            pos = end
    return res


_RANGES = {D: _tile_ranges(D) for D in (128, 32)}
_CHUNK = {128: 80, 32: 128}
_TILE_OF_EDGE = {}
_TILE_START_MASK = {}
for _D, _rng in _RANGES.items():
    toe = np.zeros(E, np.int32)
    tsm = np.zeros(E, bool)
    for _t, (_s, _e) in enumerate(_rng):
        toe[_s:_e] = _t
        tsm[_s] = True
    _TILE_OF_EDGE[_D] = toe
    _TILE_START_MASK[_D] = tsm


def _make_seg_kernel(D):
    K = _CHUNK[D]
    ranges = _RANGES[D]
    s0_np = [s // K for s, _ in ranges]
    nch_np = [(e - s) // K for s, e in ranges]
    nvec = D // 16
    out_rows = N + NTILE + 2 * NTILE

    def body(h_hbm, pack_hbm, sci_hbm, out_hbm,
             pb0, pb1, sc0, sc1, rw0, rw1, sg0, sg1,
             ps0, ps1, gs0, gs1, ss0, ss1):
        packs = (pb0, pb1)
        scis = (sc0, sc1)
        rows = (rw0, rw1)
        stgs = (sg0, sg1)
        psems = (ps0, ps1)
        gsems = (gs0, gs1)
        ssems = (ss0, ss1)
        cid = lax.axis_index("c")
        sid = lax.axis_index("s")
        wid = cid * 16 + sid
        s0 = jnp.int32(0)
        nch = jnp.int32(0)
        for t in range(NTILE):
            s0 = jnp.where(wid == t, jnp.int32(s0_np[t]), s0)
            nch = jnp.where(wid == t, jnp.int32(nch_np[t]), nch)

        def issue_pack(g, b):
            off = pl.multiple_of(g * 2 * K, 8)
            off2 = pl.multiple_of(g * K, 8)
            pltpu.async_copy(pack_hbm.at[pl.ds(off, 2 * K)], packs[b], psems[b])
            pltpu.async_copy(sci_hbm.at[pl.ds(off2, K)], scis[b], psems[b])

        def wait_pack(b):
            pltpu.make_async_copy(pack_hbm.at[pl.ds(0, 2 * K)], packs[b],
                                  psems[b]).wait()
            pltpu.make_async_copy(sci_hbm.at[pl.ds(0, K)], scis[b],
                                  psems[b]).wait()

        def issue_gather(b):
            pltpu.async_copy(h_hbm.at[packs[b].at[pl.ds(0, K)]], rows[b],
                             gsems[b])

        def wait_gather(b):
            pltpu.make_async_copy(h_hbm.at[pl.ds(0, K)], rows[b],
                                  gsems[b]).wait()

        def compute(b, accs):
            pb = packs[b]
            rw = rows[b]
            sg = stgs[b]

            def group(j, accs):
                base = j * 16
                rv = pb[pl.ds(K + base, 16)]
                for u in range(16):
                    fresh = rv[u] == 1
                    new_accs = []
                    for q in range(nvec):
                        r = rw[base + u, pl.ds(q * 16, 16)]
                        a = jnp.where(fresh, r, accs[q] + r)
                        sg[base + u, pl.ds(q * 16, 16)] = a
                        new_accs.append(a)
                    accs = tuple(new_accs)
                return accs

            return lax.fori_loop(0, K // 16, group, accs)

        def wait_scatter(b):
            pltpu.make_async_copy(stgs[b], out_hbm.at[scis[b]],
                                  ssems[b]).wait()

        def slot(c, b, accs):
            @pl.when(c >= 1)
            def _():
                wait_scatter(b ^ 1)

            @pl.when(c + 1 < nch)
            def _():
                issue_pack(s0 + c + 1, b ^ 1)
            wait_gather(b)
            accs = compute(b, accs)
            pltpu.async_copy(stgs[b], out_hbm.at[scis[b]], ssems[b])

            @pl.when(c + 1 < nch)
            def _():
                wait_pack(b ^ 1)
                issue_gather(b ^ 1)
            return accs

        issue_pack(s0, 0)
        wait_pack(0)
        issue_gather(0)

        def pair(c2, accs):
            accs = slot(2 * c2, 0, accs)
            accs = slot(2 * c2 + 1, 1, accs)
            return accs

        accs = lax.fori_loop(0, nch // 2, pair,
                             tuple(jnp.zeros((16,), jnp.float32)
                                   for _ in range(nvec)))
        odd = (nch % 2) == 1

        @pl.when(odd)
        def _():
            wait_gather(0)
            a2 = compute(0, accs)
            del a2
            pltpu.sync_copy(stgs[0], out_hbm.at[scis[0]])
        wait_scatter(1)

    return pl.kernel(
        body,
        out_type=jax.ShapeDtypeStruct((out_rows, 128), jnp.float32),
        mesh=plsc.VectorSubcoreMesh(core_axis_name="c", subcore_axis_name="s"),
        scratch_types=[
            pltpu.VMEM((2 * K,), jnp.int32),
            pltpu.VMEM((2 * K,), jnp.int32),
            pltpu.VMEM((K,), jnp.int32),
            pltpu.VMEM((K,), jnp.int32),
            pltpu.VMEM((K, 128), jnp.float32),
            pltpu.VMEM((K, 128), jnp.float32),
            pltpu.VMEM((K, 128), jnp.float32),
            pltpu.VMEM((K, 128), jnp.float32),
            pltpu.SemaphoreType.DMA,
            pltpu.SemaphoreType.DMA,
            pltpu.SemaphoreType.DMA,
            pltpu.SemaphoreType.DMA,
            pltpu.SemaphoreType.DMA,
            pltpu.SemaphoreType.DMA,
        ],
    )


_SEG_KERNELS = {}


def _seg_kernel(D):
    if D not in _SEG_KERNELS:
        _SEG_KERNELS[D] = _make_seg_kernel(D)
    return _SEG_KERNELS[D]


def _edge_plan(dst_s, D):
    toe = jnp.asarray(_TILE_OF_EDGE[D])
    tsm = jnp.asarray(_TILE_START_MASK[D])
    prev_dst = jnp.concatenate([dst_s[:1] - 1, dst_s[:-1]])
    next_dst = jnp.concatenate([dst_s[1:], dst_s[-1:] - 1])
    next_tsm = jnp.concatenate([tsm[1:], jnp.ones((1,), bool)])
    run_start = (dst_s != prev_dst) | tsm
    run_end = (dst_s != next_dst) | next_tsm

    starts = jnp.asarray([s for s, _ in _RANGES[D]], jnp.int32)
    ends = jnp.asarray([e for _, e in _RANGES[D]], jnp.int32)
    first_node = dst_s[starts]
    last_node = dst_s[ends - 1]
    is_first = dst_s == first_node[toe]
    is_last = dst_s == last_node[toe]

    scidx = jnp.where(~run_end, N + toe,
             jnp.where(is_first, N + NTILE + 2 * toe,
              jnp.where(is_last, N + NTILE + 2 * toe + 1, dst_s)))

    single = first_node == last_node
    mnode = jnp.zeros((2 * NTILE,), jnp.int32)
    mnode = mnode.at[0::2].set(first_node)
    mnode = mnode.at[1::2].set(jnp.where(single, N, last_node))
    invalid = mnode >= N
    morder = jnp.argsort(invalid.astype(jnp.int32), stable=True)
    mnode_c = mnode[morder]
    valid_c = mnode_c < N
    same_c = jnp.concatenate([jnp.zeros((1,), bool),
                              (mnode_c[1:] == mnode_c[:-1]) & valid_c[1:]])
    gend = valid_c & jnp.concatenate([mnode_c[1:] != mnode_c[:-1],
                                      jnp.ones((1,), bool)])
    wids = jnp.where(gend, mnode_c, N)
    return (scidx.astype(jnp.int32), run_start.astype(jnp.int32),
            morder.astype(jnp.int32), same_c, wids.astype(jnp.int32))


def _pack_plan(src_s, plan, D):
    scidx, rs, _, _, _ = plan
    K = _CHUNK[D]
    pack = jnp.stack([src_s.reshape(-1, K), rs.reshape(-1, K)],
                     axis=1).reshape(-1)
    return pack, scidx


def _segsum_exact(h_nm, src_s, plan, node_mask, D):
    scidx, rs, morder, same_c, wids = plan
    if h_nm.shape[1] < 128:
        h_nm = jnp.pad(h_nm, ((0, 0), (0, 128 - h_nm.shape[1])))
    pack, sci = _pack_plan(src_s, plan, D)
    out_ext = _seg_kernel(D)(h_nm, pack, sci)[:, :D]
    prows = out_ext[N + NTILE:][morder]

    def step(carry, inp):
        same, row = inp
        acc = jnp.where(same, carry + row, row)
        return acc, acc

    _, accs = lax.scan(step, jnp.zeros((D,), jnp.float32), (same_c, prows))
    base = jnp.concatenate([out_ext[:N], jnp.zeros((1, D), jnp.float32)], 0)
    base = base.at[wids].set(accs, mode="drop")
    aggr = jnp.where(node_mask, base[:N], 0.0)
    return aggr


def _l0_body(ht_ref, at_ref, w_ref, b_ref, o_ref):
    t = ht_ref[...] + at_ref[...]
    y = jnp.dot(w_ref[...], t, preferred_element_type=jnp.float32)
    o_ref[...] = jnp.maximum(y + b_ref[...], 0.0)


def _bnmm_body(yt_ref, m_ref, v_ref, g_ref, be_ref, w_ref, b_ref, o_ref):
    r = lax.rsqrt(v_ref[...] + EPS)
    hh = (g_ref[...] * (yt_ref[...] - m_ref[...])) * r + be_ref[...]
    y = jnp.dot(w_ref[...], hh, preferred_element_type=jnp.float32)
    o_ref[...] = jnp.maximum(y + b_ref[...], 0.0)


def _bn_body(yt_ref, m_ref, v_ref, g_ref, be_ref, o_ref):
    r = lax.rsqrt(v_ref[...] + EPS)
    o_ref[...] = (g_ref[...] * (yt_ref[...] - m_ref[...])) * r + be_ref[...]


def _head_body(h_ref, batch_ref, l1t_ref, l1b_ref, clt_ref, clb_ref, o_ref):
    gids = lax.broadcasted_iota(jnp.int32, (NUM_GRAPHS, N), 0)
    onehot = (gids == batch_ref[...]).astype(jnp.float32)
    pooled = jnp.dot(onehot, h_ref[...], preferred_element_type=jnp.float32)
    z = jnp.maximum(jnp.dot(pooled, l1t_ref[...],
                            preferred_element_type=jnp.float32)
                    + l1b_ref[...], 0.0)
    logits = jnp.dot(z, clt_ref[...],
                     preferred_element_type=jnp.float32) + clb_ref[...]
    m = jnp.max(logits, axis=-1, keepdims=True)
    s = logits - m
    lse = jnp.log(jnp.sum(jnp.exp(s), axis=-1, keepdims=True))
    o_ref[...] = s - lse


def _tc(body, out_shape):
    return pl.pallas_call(body, out_shape=jax.ShapeDtypeStruct(out_shape,
                                                               jnp.float32))


def _mlp_chain(tt, at, layers):
    l0 = layers[0]
    yt = _tc(_l0_body, (H, N))(tt, at, l0["W"], l0["b"].reshape(H, 1))
    for li in range(1, len(layers) + 1):
        p = layers[li - 1]
        m = jnp.mean(yt, axis=1, keepdims=True)
        v = jnp.var(yt, axis=1, keepdims=True)
        g = p["gamma"].reshape(H, 1)
        be = p["beta"].reshape(H, 1)
        if li < len(layers):
            nxt = layers[li]
            yt = _tc(_bnmm_body, (H, N))(yt, m, v, g, be, nxt["W"],
                                         nxt["b"].reshape(H, 1))
        else:
            yt = _tc(_bn_body, (H, N))(yt, m, v, g, be)
    return yt


def kernel(x, edge_attr, edge_index, batch, conv_params, lin1_W, lin1_b,
           cls_W, cls_b):
    del edge_attr
    src = edge_index[0]
    dst = edge_index[1]
    perm = jnp.argsort(dst, stable=True)
    src_s = src[perm].astype(jnp.int32)
    dst_s = dst[perm].astype(jnp.int32)
    plans = {D: _edge_plan(dst_s, D) for D in (128, 32)}
    node_mask = jnp.zeros((N, 1), bool).at[dst, 0].set(True)

    h_nm = x
    for ci in range(3):
        D = 128 if ci == 0 else 32
        aggr = _segsum_exact(h_nm, src_s, plans[D], node_mask, D)
        yt = _mlp_chain(h_nm.T, aggr.T, conv_params[ci])
        h_nm = yt.T
    return _tc(_head_body, (NUM_GRAPHS, C))(
        h_nm, batch.reshape(1, N).astype(jnp.int32),
        lin1_W.T, lin1_b.reshape(1, H), cls_W.T, cls_b.reshape(1, C))
